# Initial kernel scaffold; baseline (speedup 1.0000x reference)
import jax
import jax.numpy as jnp
from jax.experimental import pallas as pl


def kernel(embeddings, vocab_embeddings):
    raise NotImplementedError("write your pallas kernel here")


# speedup vs baseline: 65.8927x; 65.8927x over previous
import jax
import jax.numpy as jnp
from jax.experimental import pallas as pl
from jax.experimental.pallas import tpu as pltpu

K = 8
BLOCK_V = 2048

_NEG_PAD = -2.0
_NEG_DONE = -3.0
_BIG_I32 = 2**30


def _l2n(x):
    n = jnp.sqrt(jnp.sum(x * x, axis=1, keepdims=True))
    return x / jnp.maximum(n, 1e-12)


def _block_topk_kernel(vocab_size, q_ref, v_ref, vals_ref, idx_ref):
    j = pl.program_id(0)
    qn = _l2n(q_ref[...]).astype(jnp.bfloat16)
    vn = _l2n(v_ref[...]).astype(jnp.bfloat16)
    sims = jax.lax.dot_general(
        qn, vn, dimension_numbers=(((1,), (1,)), ((), ())),
        preferred_element_type=jnp.float32)
    col = j * BLOCK_V + jax.lax.broadcasted_iota(jnp.int32, sims.shape, 1)
    sims = jnp.where(col >= vocab_size, _NEG_PAD, sims)
    vals, idxs = [], []
    for _ in range(K):
        m = jnp.max(sims, axis=1)
        ci = jnp.min(jnp.where(sims == m[:, None], col, _BIG_I32), axis=1)
        vals.append(m)
        idxs.append(ci)
        sims = jnp.where(col == ci[:, None], _NEG_DONE, sims)
    vals_ref[0] = jnp.stack(vals, axis=0)
    idx_ref[0] = jnp.stack(idxs, axis=0)


def _merge_kernel(nbk, cv_ref, ci_ref, vals_ref, idx_ref):
    n = cv_ref.shape[2]
    sims = cv_ref[...].reshape(nbk, n)
    cols = ci_ref[...].reshape(nbk, n)
    vals, idxs = [], []
    for _ in range(K):
        m = jnp.max(sims, axis=0)
        ci = jnp.min(jnp.where(sims == m[None, :], cols, _BIG_I32), axis=0)
        vals.append(m)
        idxs.append(ci)
        sims = jnp.where(cols == ci[None, :], _NEG_DONE, sims)
    vals_ref[...] = jnp.stack(vals, axis=0)
    idx_ref[...] = jnp.stack(idxs, axis=0)


def kernel(embeddings, vocab_embeddings):
    orig_shape = embeddings.shape[:-1]
    d = embeddings.shape[-1]
    q = embeddings.reshape(-1, d)
    n = q.shape[0]
    vocab_size = vocab_embeddings.shape[0]
    nb = (vocab_size + BLOCK_V - 1) // BLOCK_V
    vpad = nb * BLOCK_V
    v = jnp.pad(vocab_embeddings, ((0, vpad - vocab_size), (0, 0)))

    import functools
    cand_vals, cand_idx = pl.pallas_call(
        functools.partial(_block_topk_kernel, vocab_size),
        grid=(nb,),
        in_specs=[
            pl.BlockSpec((n, d), lambda j: (0, 0)),
            pl.BlockSpec((BLOCK_V, d), lambda j: (j, 0)),
        ],
        out_specs=[
            pl.BlockSpec((1, K, n), lambda j: (j, 0, 0)),
            pl.BlockSpec((1, K, n), lambda j: (j, 0, 0)),
        ],
        out_shape=[
            jax.ShapeDtypeStruct((nb, K, n), jnp.float32),
            jax.ShapeDtypeStruct((nb, K, n), jnp.int32),
        ],
        compiler_params=pltpu.CompilerParams(
            dimension_semantics=("parallel",)),
    )(q, v)

    fvals, fidx = pl.pallas_call(
        functools.partial(_merge_kernel, nb * K),
        out_shape=[
            jax.ShapeDtypeStruct((K, n), jnp.float32),
            jax.ShapeDtypeStruct((K, n), jnp.int32),
        ],
    )(cand_vals, cand_idx)

    topk_sim = fvals.T.reshape(orig_shape + (K,))
    topk_idx = fidx.T.reshape(orig_shape + (K,))
    token_ids = fidx[0].reshape(orig_shape)
    return (token_ids, topk_sim, topk_idx)
